# baseline (speedup 1.0000x reference)
import functools
import jax
import jax.numpy as jnp
from jax import lax
from jax.experimental import pallas as pl
from jax.experimental.pallas import tpu as pltpu, tpu_sc as plsc

N = 10000
D = 128
E = 320000
NW = 32
CHUNK = 128
NCH = 80
EP = NW * NCH * CHUNK
NA = 10112
SLAB = NA // 16
NH = 10240
HSLAB = NH // 16

_mesh = plsc.VectorSubcoreMesh(core_axis_name="c", subcore_axis_name="s")


@functools.partial(
    pl.kernel,
    out_type=jax.ShapeDtypeStruct((2, 2, NH, D), jnp.float32),
    mesh=_mesh,
    scratch_types=[
        pltpu.VMEM((NCH, CHUNK), jnp.int32),
        pltpu.VMEM((NCH, CHUNK), jnp.int32),
        pltpu.VMEM((CHUNK, D), jnp.float32),
        pltpu.VMEM_SHARED((NH, D), jnp.float32),
        pltpu.SemaphoreType.DMA,
    ],
)
def _sc_degrees(srcb, dstb, ones_hbm, zrows_hbm, out, sidx, didx, ones_v,
                hist, sem):
    c = lax.axis_index("c")
    s = lax.axis_index("s")
    b = c * 16 + s
    pltpu.sync_copy(srcb.at[b], sidx)
    pltpu.sync_copy(dstb.at[b], didx)
    pltpu.sync_copy(ones_hbm, ones_v)
    base = s * HSLAB
    pltpu.sync_copy(zrows_hbm, hist.at[pl.ds(base, HSLAB)])
    plsc.subcore_barrier()

    def body_s(j, _):
        pltpu.sync_copy(ones_v, hist.at[sidx.at[j]], add=True)
        return _

    lax.fori_loop(0, NCH, body_s, None)
    plsc.subcore_barrier()
    pltpu.sync_copy(hist.at[pl.ds(base, HSLAB)],
                    out.at[c, 0, pl.ds(base, HSLAB)])
    pltpu.sync_copy(zrows_hbm, hist.at[pl.ds(base, HSLAB)])
    plsc.subcore_barrier()

    def body_d(j, _):
        pltpu.sync_copy(ones_v, hist.at[didx.at[j]], add=True)
        return _

    lax.fori_loop(0, NCH, body_d, None)
    plsc.subcore_barrier()
    pltpu.sync_copy(hist.at[pl.ds(base, HSLAB)],
                    out.at[c, 1, pl.ds(base, HSLAB)])


@functools.partial(
    pl.kernel,
    out_type=jax.ShapeDtypeStruct((2, NA, D), jnp.float32),
    mesh=_mesh,
    scratch_types=[
        pltpu.VMEM((2, NCH, CHUNK), jnp.int32),
        pltpu.VMEM((CHUNK, D), jnp.float32),
        pltpu.VMEM_SHARED((NA, D), jnp.float32),
        pltpu.SemaphoreType.DMA((2,)),
    ],
)
def _sc_spmm(g_hbm, combb, zrows_hbm, out, cidx, rows, agg, sem):
    c = lax.axis_index("c")
    s = lax.axis_index("s")
    b = c * 16 + s
    pltpu.sync_copy(combb.at[b], cidx)
    base = s * SLAB
    pltpu.sync_copy(zrows_hbm, agg.at[pl.ds(base, SLAB)])
    plsc.subcore_barrier()

    def body(j, _):
        p = lax.rem(j, 2)
        pltpu.async_copy(g_hbm.at[cidx.at[0, j]], rows, sem.at[p]).wait()
        pltpu.sync_copy(rows, agg.at[cidx.at[1, j]], add=True)
        return _

    lax.fori_loop(0, NCH, body, None)
    plsc.subcore_barrier()
    pltpu.sync_copy(agg.at[pl.ds(base, SLAB)], out.at[c, pl.ds(base, SLAB)])


_BR = 632
_GRID = NA // _BR


def _tc_first_body(x_ref, ns_ref, w_ref, o_ref):
    o_ref[...] = jnp.dot(x_ref[...] * ns_ref[...], w_ref[...],
                         preferred_element_type=jnp.float32)


def _tc_mid_body(p_ref, nd_ref, ns_ref, b_ref, w_ref, o_ref):
    h = (p_ref[0] + p_ref[1]) * nd_ref[...] + b_ref[...]
    h = jnp.maximum(h, 0.0) * ns_ref[...]
    o_ref[...] = jnp.dot(h, w_ref[...], preferred_element_type=jnp.float32)


def _tc_final_body(p_ref, nd_ref, b_ref, o_ref):
    o_ref[...] = (p_ref[0] + p_ref[1]) * nd_ref[...] + b_ref[...]


_row_spec = pl.BlockSpec((_BR, D), lambda i: (i, 0))
_p_spec = pl.BlockSpec((2, _BR, D), lambda i: (0, i, 0))
_w_spec = pl.BlockSpec((D, D), lambda i: (0, 0))
_b_spec = pl.BlockSpec((1, D), lambda i: (0, 0))

_tc_first = pl.pallas_call(
    _tc_first_body, grid=(_GRID,),
    in_specs=[_row_spec, _row_spec, _w_spec],
    out_specs=_row_spec,
    out_shape=jax.ShapeDtypeStruct((NA, D), jnp.float32),
)

_tc_mid = pl.pallas_call(
    _tc_mid_body, grid=(_GRID,),
    in_specs=[_p_spec, _row_spec, _row_spec, _b_spec, _w_spec],
    out_specs=_row_spec,
    out_shape=jax.ShapeDtypeStruct((NA, D), jnp.float32),
)

_tc_final = pl.pallas_call(
    _tc_final_body, grid=(_GRID,),
    in_specs=[_p_spec, _row_spec, _b_spec],
    out_specs=_row_spec,
    out_shape=jax.ShapeDtypeStruct((NA, D), jnp.float32),
)


def kernel(x, edge_index, W1, b1, W2, b2, W3, b3):
    src = edge_index[0]
    dst = edge_index[1]
    npad = EP - E
    ar = jnp.arange(npad, dtype=jnp.int32)
    padh = (N + (ar % (NH - N))).astype(jnp.int32)
    srcb = jnp.concatenate([src, padh]).reshape(NW, NCH, CHUNK)
    dstb = jnp.concatenate([dst, padh]).reshape(NW, NCH, CHUNK)
    pads = jnp.zeros((npad,), jnp.int32)
    padd = (N + (ar % (NA - N))).astype(jnp.int32)
    src2 = jnp.concatenate([src, pads]).reshape(NW, NCH, CHUNK)
    dst2 = jnp.concatenate([dst, padd]).reshape(NW, NCH, CHUNK)
    comb = jnp.stack([src2, dst2], axis=1)

    ones_rows = jnp.ones((CHUNK, D), jnp.float32)
    zrows_h = jnp.zeros((HSLAB, D), jnp.float32)
    zrows_a = jnp.zeros((SLAB, D), jnp.float32)

    hist = _sc_degrees(srcb, dstb, ones_rows, zrows_h)
    deg = hist.sum(axis=0)[:, :NA, 0]
    norm = jnp.where(deg > 0, lax.rsqrt(jnp.maximum(deg, 1.0)), 0.0)
    ns = jnp.broadcast_to(norm[0][:, None], (NA, D))
    nd = jnp.broadcast_to(norm[1][:, None], (NA, D))
    x_pad = jnp.zeros((NA, D), x.dtype).at[:N].set(x)

    b1r = b1.reshape(1, D)
    b2r = b2.reshape(1, D)
    b3r = b3.reshape(1, D)

    g = _tc_first(x_pad, ns, W1)
    p = _sc_spmm(g, comb, zrows_a)
    g = _tc_mid(p, nd, ns, b1r, W2)
    p = _sc_spmm(g, comb, zrows_a)
    g = _tc_mid(p, nd, ns, b2r, W3)
    p = _sc_spmm(g, comb, zrows_a)
    return _tc_final(p, nd, b3r)[:N]


# speedup vs baseline: 1.0623x; 1.0623x over previous
import functools
import jax
import jax.numpy as jnp
from jax import lax
from jax.experimental import pallas as pl
from jax.experimental.pallas import tpu as pltpu, tpu_sc as plsc

N = 10000
D = 128
E = 320000
NW = 32
CHUNK = 128
NCH = 80
EP = NW * NCH * CHUNK
NA = 10112
SLAB = NA // 16
NH = 10240
HSLAB = NH // 16

_mesh = plsc.VectorSubcoreMesh(core_axis_name="c", subcore_axis_name="s")


@functools.partial(
    pl.kernel,
    out_type=jax.ShapeDtypeStruct((2, 2, NH, D), jnp.float32),
    mesh=_mesh,
    scratch_types=[
        pltpu.VMEM((NCH, CHUNK), jnp.int32),
        pltpu.VMEM((NCH, CHUNK), jnp.int32),
        pltpu.VMEM((CHUNK, D), jnp.float32),
        pltpu.VMEM_SHARED((NH, D), jnp.float32),
        pltpu.SemaphoreType.DMA,
    ],
)
def _sc_degrees(srcb, dstb, ones_hbm, zrows_hbm, out, sidx, didx, ones_v,
                hist, sem):
    c = lax.axis_index("c")
    s = lax.axis_index("s")
    b = c * 16 + s
    pltpu.sync_copy(srcb.at[b], sidx)
    pltpu.sync_copy(dstb.at[b], didx)
    pltpu.sync_copy(ones_hbm, ones_v)
    base = s * HSLAB
    pltpu.sync_copy(zrows_hbm, hist.at[pl.ds(base, HSLAB)])
    plsc.subcore_barrier()

    def body_s(j, _):
        pltpu.sync_copy(ones_v, hist.at[sidx.at[j]], add=True)
        return _

    lax.fori_loop(0, NCH, body_s, None)
    plsc.subcore_barrier()
    pltpu.sync_copy(hist.at[pl.ds(base, HSLAB)],
                    out.at[c, 0, pl.ds(base, HSLAB)])
    pltpu.sync_copy(zrows_hbm, hist.at[pl.ds(base, HSLAB)])
    plsc.subcore_barrier()

    def body_d(j, _):
        pltpu.sync_copy(ones_v, hist.at[didx.at[j]], add=True)
        return _

    lax.fori_loop(0, NCH, body_d, None)
    plsc.subcore_barrier()
    pltpu.sync_copy(hist.at[pl.ds(base, HSLAB)],
                    out.at[c, 1, pl.ds(base, HSLAB)])


@functools.partial(
    pl.kernel,
    out_type=jax.ShapeDtypeStruct((2, NA, D), jnp.float32),
    mesh=_mesh,
    scratch_types=[
        pltpu.VMEM((NCH, CHUNK), jnp.int32),
        pltpu.VMEM((NCH, CHUNK), jnp.int32),
        pltpu.VMEM((CHUNK, D), jnp.float32),
        pltpu.VMEM_SHARED((NA, D), jnp.float32),
        pltpu.SemaphoreType.DMA,
    ],
)
def _sc_spmm(g_hbm, srcb, dstb, zrows_hbm, out, sidx, didx, rows, agg, sem):
    c = lax.axis_index("c")
    s = lax.axis_index("s")
    b = c * 16 + s
    pltpu.sync_copy(srcb.at[b], sidx)
    pltpu.sync_copy(dstb.at[b], didx)
    base = s * SLAB
    pltpu.sync_copy(zrows_hbm, agg.at[pl.ds(base, SLAB)])
    plsc.subcore_barrier()

    def body(j, _):
        pltpu.async_copy(g_hbm.at[sidx.at[j]], rows, sem).wait()
        pltpu.sync_copy(rows, agg.at[didx.at[j]], add=True)
        return _

    lax.fori_loop(0, NCH, body, None)
    plsc.subcore_barrier()
    pltpu.sync_copy(agg.at[pl.ds(base, SLAB)], out.at[c, pl.ds(base, SLAB)])


_BR = 632
_GRID = NA // _BR


def _tc_first_body(x_ref, ns_ref, w_ref, o_ref):
    o_ref[...] = jnp.dot(x_ref[...] * ns_ref[...], w_ref[...],
                         preferred_element_type=jnp.float32)


def _tc_mid_body(p_ref, nd_ref, ns_ref, b_ref, w_ref, o_ref):
    h = (p_ref[0] + p_ref[1]) * nd_ref[...] + b_ref[...]
    h = jnp.maximum(h, 0.0) * ns_ref[...]
    o_ref[...] = jnp.dot(h, w_ref[...], preferred_element_type=jnp.float32)


def _tc_final_body(p_ref, nd_ref, b_ref, o_ref):
    o_ref[...] = (p_ref[0] + p_ref[1]) * nd_ref[...] + b_ref[...]


_row_spec = pl.BlockSpec((_BR, D), lambda i: (i, 0))
_p_spec = pl.BlockSpec((2, _BR, D), lambda i: (0, i, 0))
_w_spec = pl.BlockSpec((D, D), lambda i: (0, 0))
_b_spec = pl.BlockSpec((1, D), lambda i: (0, 0))

_tc_first = pl.pallas_call(
    _tc_first_body, grid=(_GRID,),
    in_specs=[_row_spec, _row_spec, _w_spec],
    out_specs=_row_spec,
    out_shape=jax.ShapeDtypeStruct((NA, D), jnp.float32),
)

_tc_mid = pl.pallas_call(
    _tc_mid_body, grid=(_GRID,),
    in_specs=[_p_spec, _row_spec, _row_spec, _b_spec, _w_spec],
    out_specs=_row_spec,
    out_shape=jax.ShapeDtypeStruct((NA, D), jnp.float32),
)

_tc_final = pl.pallas_call(
    _tc_final_body, grid=(_GRID,),
    in_specs=[_p_spec, _row_spec, _b_spec],
    out_specs=_row_spec,
    out_shape=jax.ShapeDtypeStruct((NA, D), jnp.float32),
)


def kernel(x, edge_index, W1, b1, W2, b2, W3, b3):
    src = edge_index[0]
    dst = edge_index[1]
    npad = EP - E
    ar = jnp.arange(npad, dtype=jnp.int32)
    padh = (N + (ar % (NH - N))).astype(jnp.int32)
    srcb = jnp.concatenate([src, padh]).reshape(NW, NCH, CHUNK)
    dstb = jnp.concatenate([dst, padh]).reshape(NW, NCH, CHUNK)
    pads = jnp.zeros((npad,), jnp.int32)
    padd = (N + (ar % (NA - N))).astype(jnp.int32)
    src2 = jnp.concatenate([src, pads]).reshape(NW, NCH, CHUNK)
    dst2 = jnp.concatenate([dst, padd]).reshape(NW, NCH, CHUNK)
    ones_rows = jnp.ones((CHUNK, D), jnp.float32)
    zrows_h = jnp.zeros((HSLAB, D), jnp.float32)
    zrows_a = jnp.zeros((SLAB, D), jnp.float32)

    hist = _sc_degrees(srcb, dstb, ones_rows, zrows_h)
    deg = hist.sum(axis=0)[:, :NA, 0]
    norm = jnp.where(deg > 0, lax.rsqrt(jnp.maximum(deg, 1.0)), 0.0)
    ns = jnp.broadcast_to(norm[0][:, None], (NA, D))
    nd = jnp.broadcast_to(norm[1][:, None], (NA, D))
    x_pad = jnp.zeros((NA, D), x.dtype).at[:N].set(x)

    b1r = b1.reshape(1, D)
    b2r = b2.reshape(1, D)
    b3r = b3.reshape(1, D)

    g = _tc_first(x_pad, ns, W1)
    p = _sc_spmm(g, src2, dst2, zrows_a)
    g = _tc_mid(p, nd, ns, b1r, W2)
    p = _sc_spmm(g, src2, dst2, zrows_a)
    g = _tc_mid(p, nd, ns, b2r, W3)
    p = _sc_spmm(g, src2, dst2, zrows_a)
    return _tc_final(p, nd, b3r)[:N]


# speedup vs baseline: 2.4339x; 2.2912x over previous
import functools
import jax
import jax.numpy as jnp
from jax import lax
from jax.experimental import pallas as pl
from jax.experimental.pallas import tpu as pltpu, tpu_sc as plsc

N = 10000
D = 128
E = 320000
NW = 32
CHUNK = 128
NCH = 80
EP = NW * NCH * CHUNK
NA = 10112
SLAB = NA // 16
NH = 10240
HSLAB = NH // 16

_mesh = plsc.VectorSubcoreMesh(core_axis_name="c", subcore_axis_name="s")


@functools.partial(
    pl.kernel,
    out_type=jax.ShapeDtypeStruct((2, 2, NH, D), jnp.float32),
    mesh=_mesh,
    scratch_types=[
        pltpu.VMEM((NCH, CHUNK), jnp.int32),
        pltpu.VMEM((NCH, CHUNK), jnp.int32),
        pltpu.VMEM((CHUNK, D), jnp.float32),
        pltpu.VMEM_SHARED((NH, D), jnp.float32),
        pltpu.SemaphoreType.DMA,
    ],
)
def _sc_degrees(srcb, dstb, ones_hbm, zrows_hbm, out, sidx, didx, ones_v,
                hist, sem):
    c = lax.axis_index("c")
    s = lax.axis_index("s")
    b = c * 16 + s
    pltpu.sync_copy(srcb.at[b], sidx)
    pltpu.sync_copy(dstb.at[b], didx)
    pltpu.sync_copy(ones_hbm, ones_v)
    base = s * HSLAB
    pltpu.sync_copy(zrows_hbm, hist.at[pl.ds(base, HSLAB)])
    plsc.subcore_barrier()

    def body_s(j, _):
        pltpu.sync_copy(ones_v, hist.at[sidx.at[j]], add=True)
        return _

    lax.fori_loop(0, NCH, body_s, None)
    plsc.subcore_barrier()
    pltpu.sync_copy(hist.at[pl.ds(base, HSLAB)],
                    out.at[c, 0, pl.ds(base, HSLAB)])
    pltpu.sync_copy(zrows_hbm, hist.at[pl.ds(base, HSLAB)])
    plsc.subcore_barrier()

    def body_d(j, _):
        pltpu.sync_copy(ones_v, hist.at[didx.at[j]], add=True)
        return _

    lax.fori_loop(0, NCH, body_d, None)
    plsc.subcore_barrier()
    pltpu.sync_copy(hist.at[pl.ds(base, HSLAB)],
                    out.at[c, 1, pl.ds(base, HSLAB)])


@functools.partial(
    pl.kernel,
    out_type=jax.ShapeDtypeStruct((2, NA, D), jnp.float32),
    mesh=_mesh,
    scratch_types=[
        pltpu.VMEM((NCH, CHUNK), jnp.int32),
        pltpu.VMEM((NCH, CHUNK), jnp.int32),
        pltpu.VMEM((CHUNK, D), jnp.float32),
        pltpu.VMEM_SHARED((NA, D), jnp.float32),
        pltpu.SemaphoreType.DMA,
    ],
)
def _sc_spmm(g_hbm, srcb, dstb, zrows_hbm, out, sidx, didx, rows, agg, sem):
    c = lax.axis_index("c")
    s = lax.axis_index("s")
    b = c * 16 + s
    pltpu.sync_copy(srcb.at[b], sidx)
    pltpu.sync_copy(dstb.at[b], didx)
    base = s * SLAB
    pltpu.sync_copy(zrows_hbm, agg.at[pl.ds(base, SLAB)])
    plsc.subcore_barrier()

    def body(j, _):
        pltpu.async_copy(g_hbm.at[sidx.at[j]], rows, sem).wait()
        pltpu.sync_copy(rows, agg.at[didx.at[j]], add=True)
        return _

    lax.fori_loop(0, NCH, body, None)
    plsc.subcore_barrier()
    pltpu.sync_copy(agg.at[pl.ds(base, SLAB)], out.at[c, pl.ds(base, SLAB)])


_BR = 632
_GRID = NA // _BR


def _tc_first_body(x_ref, ns_ref, w_ref, o_ref):
    o_ref[...] = jnp.dot(x_ref[...] * ns_ref[...], w_ref[...],
                         preferred_element_type=jnp.float32)


def _tc_mid_body(p_ref, nd_ref, ns_ref, b_ref, w_ref, o_ref):
    h = (p_ref[0] + p_ref[1]) * nd_ref[...] + b_ref[...]
    h = jnp.maximum(h, 0.0) * ns_ref[...]
    o_ref[...] = jnp.dot(h, w_ref[...], preferred_element_type=jnp.float32)


def _tc_final_body(p_ref, nd_ref, b_ref, o_ref):
    o_ref[...] = (p_ref[0] + p_ref[1]) * nd_ref[...] + b_ref[...]


_row_spec = pl.BlockSpec((_BR, D), lambda i: (i, 0))
_p_spec = pl.BlockSpec((2, _BR, D), lambda i: (0, i, 0))
_w_spec = pl.BlockSpec((D, D), lambda i: (0, 0))
_b_spec = pl.BlockSpec((1, D), lambda i: (0, 0))

_tc_first = pl.pallas_call(
    _tc_first_body, grid=(_GRID,),
    in_specs=[_row_spec, _row_spec, _w_spec],
    out_specs=_row_spec,
    out_shape=jax.ShapeDtypeStruct((NA, D), jnp.float32),
)

_tc_mid = pl.pallas_call(
    _tc_mid_body, grid=(_GRID,),
    in_specs=[_p_spec, _row_spec, _row_spec, _b_spec, _w_spec],
    out_specs=_row_spec,
    out_shape=jax.ShapeDtypeStruct((NA, D), jnp.float32),
)

_tc_final = pl.pallas_call(
    _tc_final_body, grid=(_GRID,),
    in_specs=[_p_spec, _row_spec, _b_spec],
    out_specs=_row_spec,
    out_shape=jax.ShapeDtypeStruct((NA, D), jnp.float32),
)


def kernel(x, edge_index, W1, b1, W2, b2, W3, b3):
    src = edge_index[0]
    dst = edge_index[1]
    npad = EP - E
    ar = jnp.arange(npad, dtype=jnp.int32)
    padh = (N + (ar % (NH - N))).astype(jnp.int32)
    srcb = jnp.concatenate([src, padh]).reshape(NW, NCH, CHUNK)
    dstb = jnp.concatenate([dst, padh]).reshape(NW, NCH, CHUNK)
    pads = (ar * 521) % N
    padd = (N + (ar % (NA - N))).astype(jnp.int32)
    src2 = jnp.concatenate([src, pads]).reshape(NW, NCH, CHUNK)
    dst2 = jnp.concatenate([dst, padd]).reshape(NW, NCH, CHUNK)
    ones_rows = jnp.ones((CHUNK, D), jnp.float32)
    zrows_h = jnp.zeros((HSLAB, D), jnp.float32)
    zrows_a = jnp.zeros((SLAB, D), jnp.float32)

    hist = _sc_degrees(srcb, dstb, ones_rows, zrows_h)
    deg = hist.sum(axis=0)[:, :NA, 0]
    norm = jnp.where(deg > 0, lax.rsqrt(jnp.maximum(deg, 1.0)), 0.0)
    ns = jnp.broadcast_to(norm[0][:, None], (NA, D))
    nd = jnp.broadcast_to(norm[1][:, None], (NA, D))
    x_pad = jnp.zeros((NA, D), x.dtype).at[:N].set(x)

    b1r = b1.reshape(1, D)
    b2r = b2.reshape(1, D)
    b3r = b3.reshape(1, D)

    g = _tc_first(x_pad, ns, W1)
    p = _sc_spmm(g, src2, dst2, zrows_a)
    g = _tc_mid(p, nd, ns, b1r, W2)
    p = _sc_spmm(g, src2, dst2, zrows_a)
    g = _tc_mid(p, nd, ns, b2r, W3)
    p = _sc_spmm(g, src2, dst2, zrows_a)
    return _tc_final(p, nd, b3r)[:N]


# speedup vs baseline: 2.7262x; 1.1201x over previous
import functools
import jax
import jax.numpy as jnp
from jax import lax
from jax.experimental import pallas as pl
from jax.experimental.pallas import tpu as pltpu, tpu_sc as plsc

N = 10000
D = 128
E = 320000
NW = 32
CHUNK = 128
NCH = 80
EP = NW * NCH * CHUNK
NA = 10112
SLAB = NA // 16
NH = 10240
DW = 16
HSLAB = NH // 16

_mesh = plsc.VectorSubcoreMesh(core_axis_name="c", subcore_axis_name="s")


@functools.partial(
    pl.kernel,
    out_type=jax.ShapeDtypeStruct((2, 2, NH, DW), jnp.float32),
    mesh=_mesh,
    scratch_types=[
        pltpu.VMEM((NCH, CHUNK), jnp.int32),
        pltpu.VMEM((NCH, CHUNK), jnp.int32),
        pltpu.VMEM((CHUNK, DW), jnp.float32),
        pltpu.VMEM_SHARED((NH, DW), jnp.float32),
    ],
)
def _sc_degrees(srcb, dstb, ones_hbm, zrows_hbm, out, sidx, didx, ones_v,
                hist):
    c = lax.axis_index("c")
    s = lax.axis_index("s")
    b = c * 16 + s
    pltpu.sync_copy(srcb.at[b], sidx)
    pltpu.sync_copy(dstb.at[b], didx)
    pltpu.sync_copy(ones_hbm, ones_v)
    base = s * HSLAB
    pltpu.sync_copy(zrows_hbm, hist.at[pl.ds(base, HSLAB)])
    plsc.subcore_barrier()

    def body_s(j, _):
        pltpu.sync_copy(ones_v, hist.at[sidx.at[j]], add=True)
        return _

    lax.fori_loop(0, NCH, body_s, None)
    plsc.subcore_barrier()
    pltpu.sync_copy(hist.at[pl.ds(base, HSLAB)],
                    out.at[c, 0, pl.ds(base, HSLAB)])
    pltpu.sync_copy(zrows_hbm, hist.at[pl.ds(base, HSLAB)])
    plsc.subcore_barrier()

    def body_d(j, _):
        pltpu.sync_copy(ones_v, hist.at[didx.at[j]], add=True)
        return _

    lax.fori_loop(0, NCH, body_d, None)
    plsc.subcore_barrier()
    pltpu.sync_copy(hist.at[pl.ds(base, HSLAB)],
                    out.at[c, 1, pl.ds(base, HSLAB)])


@functools.partial(
    pl.kernel,
    out_type=jax.ShapeDtypeStruct((2, NA, D), jnp.float32),
    mesh=_mesh,
    scratch_types=[
        pltpu.VMEM((NCH, CHUNK), jnp.int32),
        pltpu.VMEM((NCH, CHUNK), jnp.int32),
        pltpu.VMEM((CHUNK, D), jnp.float32),
        pltpu.VMEM_SHARED((NA, D), jnp.float32),
        pltpu.SemaphoreType.DMA,
    ],
)
def _sc_spmm(g_hbm, srcb, dstb, zrows_hbm, out, sidx, didx, rows, agg, sem):
    c = lax.axis_index("c")
    s = lax.axis_index("s")
    b = c * 16 + s
    pltpu.sync_copy(srcb.at[b], sidx)
    pltpu.sync_copy(dstb.at[b], didx)
    base = s * SLAB
    pltpu.sync_copy(zrows_hbm, agg.at[pl.ds(base, SLAB)])
    plsc.subcore_barrier()

    def body(j, _):
        pltpu.async_copy(g_hbm.at[sidx.at[j]], rows, sem).wait()
        pltpu.sync_copy(rows, agg.at[didx.at[j]], add=True)
        return _

    lax.fori_loop(0, NCH, body, None)
    plsc.subcore_barrier()
    pltpu.sync_copy(agg.at[pl.ds(base, SLAB)], out.at[c, pl.ds(base, SLAB)])


_BR = 632
_GRID = NA // _BR


def _tc_first_body(x_ref, ns_ref, w_ref, o_ref):
    o_ref[...] = jnp.dot(x_ref[...] * ns_ref[...], w_ref[...],
                         preferred_element_type=jnp.float32)


def _tc_mid_body(p_ref, nd_ref, ns_ref, b_ref, w_ref, o_ref):
    h = (p_ref[0] + p_ref[1]) * nd_ref[...] + b_ref[...]
    h = jnp.maximum(h, 0.0) * ns_ref[...]
    o_ref[...] = jnp.dot(h, w_ref[...], preferred_element_type=jnp.float32)


def _tc_final_body(p_ref, nd_ref, b_ref, o_ref):
    o_ref[...] = (p_ref[0] + p_ref[1]) * nd_ref[...] + b_ref[...]


_row_spec = pl.BlockSpec((_BR, D), lambda i: (i, 0))
_p_spec = pl.BlockSpec((2, _BR, D), lambda i: (0, i, 0))
_w_spec = pl.BlockSpec((D, D), lambda i: (0, 0))
_b_spec = pl.BlockSpec((1, D), lambda i: (0, 0))

_tc_first = pl.pallas_call(
    _tc_first_body, grid=(_GRID,),
    in_specs=[_row_spec, _row_spec, _w_spec],
    out_specs=_row_spec,
    out_shape=jax.ShapeDtypeStruct((NA, D), jnp.float32),
)

_tc_mid = pl.pallas_call(
    _tc_mid_body, grid=(_GRID,),
    in_specs=[_p_spec, _row_spec, _row_spec, _b_spec, _w_spec],
    out_specs=_row_spec,
    out_shape=jax.ShapeDtypeStruct((NA, D), jnp.float32),
)

_tc_final = pl.pallas_call(
    _tc_final_body, grid=(_GRID,),
    in_specs=[_p_spec, _row_spec, _b_spec],
    out_specs=_row_spec,
    out_shape=jax.ShapeDtypeStruct((NA, D), jnp.float32),
)


def kernel(x, edge_index, W1, b1, W2, b2, W3, b3):
    src = edge_index[0]
    dst = edge_index[1]
    npad = EP - E
    ar = jnp.arange(npad, dtype=jnp.int32)
    padh = (N + (ar % (NH - N))).astype(jnp.int32)
    srcb = jnp.concatenate([src, padh]).reshape(NW, NCH, CHUNK)
    dstb = jnp.concatenate([dst, padh]).reshape(NW, NCH, CHUNK)
    pads = (ar * 521) % N
    padd = (N + (ar % (NA - N))).astype(jnp.int32)
    src2 = jnp.concatenate([src, pads]).reshape(NW, NCH, CHUNK)
    dst2 = jnp.concatenate([dst, padd]).reshape(NW, NCH, CHUNK)
    ones_rows = jnp.ones((CHUNK, DW), jnp.float32)
    zrows_h = jnp.zeros((HSLAB, DW), jnp.float32)
    zrows_a = jnp.zeros((SLAB, D), jnp.float32)

    hist = _sc_degrees(srcb, dstb, ones_rows, zrows_h)
    deg = hist.sum(axis=0)[:, :NA, 0]
    norm = jnp.where(deg > 0, lax.rsqrt(jnp.maximum(deg, 1.0)), 0.0)
    ns = jnp.broadcast_to(norm[0][:, None], (NA, D))
    nd = jnp.broadcast_to(norm[1][:, None], (NA, D))
    x_pad = jnp.zeros((NA, D), x.dtype).at[:N].set(x)

    b1r = b1.reshape(1, D)
    b2r = b2.reshape(1, D)
    b3r = b3.reshape(1, D)

    g = _tc_first(x_pad, ns, W1)
    p = _sc_spmm(g, src2, dst2, zrows_a)
    g = _tc_mid(p, nd, ns, b1r, W2)
    p = _sc_spmm(g, src2, dst2, zrows_a)
    g = _tc_mid(p, nd, ns, b2r, W3)
    p = _sc_spmm(g, src2, dst2, zrows_a)
    return _tc_final(p, nd, b3r)[:N]
